# Initial kernel scaffold; baseline (speedup 1.0000x reference)
import jax
import jax.numpy as jnp
from jax.experimental import pallas as pl


def kernel(E_u_0, E_i_0, Wu, bu, Wi, bi, pos_values, neg_values, pos_edge_index, neg_edge_index, uids, iids, labels):
    raise NotImplementedError("write your pallas kernel here")


# speedup vs baseline: 2.7810x; 2.7810x over previous
import functools

import jax
import jax.numpy as jnp
from jax import lax
from jax.experimental import pallas as pl
from jax.experimental.pallas import tpu as pltpu
from jax.experimental.pallas import tpu_sc as plsc

NU = 10000
NI = 10000
DIM = 128
NLAYER = 3
NE = 320000
NB = 16384

NCORE = 2
NSUB = 16
LANE = 16

CHUNK = 80
EPT = NE // NSUB
NCHUNK = EPT // CHUNK
NUP = 10240
ZROWS = 128
RPT = NUP // NSUB

_sc_mesh = plsc.VectorSubcoreMesh(core_axis_name="c", subcore_axis_name="s")


def _sc_spmm_body(ecat, srcs, dsts, vals, out, src_v, dst_v, val_v, rows_v,
                  zero_v, accum, sem):
    cid = lax.axis_index("c")
    sid = lax.axis_index("s")
    zv = jnp.zeros((LANE,), jnp.float32)
    for r in range(ZROWS):
        for j in range(DIM // LANE):
            zero_v[r, pl.ds(j * LANE, LANE)] = zv
    row0 = sid * RPT

    def run_phase(s, carry):
        p = cid * 2 + s
        for k in range(RPT // ZROWS):
            pltpu.sync_copy(zero_v, accum.at[pl.ds(row0 + k * ZROWS, ZROWS)])
        plsc.subcore_barrier()
        ebase = p * NE + sid * EPT

        def chunk(i, c2):
            off = ebase + i * CHUNK
            pltpu.sync_copy(srcs.at[pl.ds(off, CHUNK)], src_v)
            pltpu.sync_copy(dsts.at[pl.ds(off, CHUNK)], dst_v)
            pltpu.sync_copy(vals.at[pl.ds(off, CHUNK)], val_v)
            pltpu.async_copy(ecat.at[src_v], rows_v, sem).wait()
            for g in range(CHUNK // LANE):
                vv = val_v[pl.ds(g * LANE, LANE)]
                for t in range(LANE):
                    e = g * LANE + t
                    bv = vv[t]
                    for j in range(DIM // LANE):
                        sl = pl.ds(j * LANE, LANE)
                        rows_v[e, sl] = rows_v[e, sl] * bv
            pltpu.sync_copy(rows_v, accum.at[dst_v], add=True)
            return c2

        lax.fori_loop(0, NCHUNK, chunk, 0)
        plsc.subcore_barrier()
        for k in range(RPT // ZROWS):
            r0 = row0 + k * ZROWS
            pltpu.sync_copy(accum.at[pl.ds(r0, ZROWS)],
                            out.at[p, pl.ds(r0, ZROWS)])
        return carry

    lax.fori_loop(0, 2, run_phase, 0)


_sc_spmm = functools.partial(
    pl.kernel,
    _sc_spmm_body,
    mesh=_sc_mesh,
    out_type=jax.ShapeDtypeStruct((4, NUP, DIM), jnp.float32),
    scratch_types=[
        pltpu.VMEM((CHUNK,), jnp.int32),
        pltpu.VMEM((CHUNK,), jnp.int32),
        pltpu.VMEM((CHUNK,), jnp.float32),
        pltpu.VMEM((CHUNK, DIM), jnp.float32),
        pltpu.VMEM((ZROWS, DIM), jnp.float32),
        pltpu.VMEM_SHARED((NUP, DIM), jnp.float32),
        pltpu.SemaphoreType.DMA,
    ],
)()


GPT = 2 * NB // (NCORE * NSUB)
GC = 128


def _sc_gather_body(ecat, idx, out, idx_v, rows_v, sem):
    cid = lax.axis_index("c")
    sid = lax.axis_index("s")
    wid = sid * NCORE + cid
    base = wid * GPT

    def chunk(i, c):
        off = base + i * GC
        pltpu.sync_copy(idx.at[pl.ds(off, GC)], idx_v)
        pltpu.async_copy(ecat.at[idx_v], rows_v, sem).wait()
        pltpu.sync_copy(rows_v, out.at[pl.ds(off, GC)])
        return c

    lax.fori_loop(0, GPT // GC, chunk, 0)


_sc_gather = functools.partial(
    pl.kernel,
    _sc_gather_body,
    mesh=_sc_mesh,
    out_type=jax.ShapeDtypeStruct((2 * NB, DIM), jnp.float32),
    scratch_types=[
        pltpu.VMEM((GC,), jnp.int32),
        pltpu.VMEM((GC, DIM), jnp.float32),
        pltpu.SemaphoreType.DMA,
    ],
)()


RB = 1000


def _tc_transform_body(x_ref, zp_ref, zn_ref, w_ref, b_ref, o_ref):
    x = x_ref[...]
    zp = zp_ref[0]
    zn = zn_ref[0]
    w = w_ref[0]
    acc = jnp.dot(x, w[0].T, preferred_element_type=jnp.float32)
    acc += jnp.dot(zp, w[1].T, preferred_element_type=jnp.float32)
    acc += jnp.dot(zp * x, w[2].T, preferred_element_type=jnp.float32)
    acc += jnp.dot(zn, w[3].T, preferred_element_type=jnp.float32)
    acc += jnp.dot(zn * x, w[4].T, preferred_element_type=jnp.float32)
    acc += b_ref[0]
    o_ref[...] = jnp.where(acc >= 0.0, acc, 0.2 * acc)


def _tc_transform(ecat, zcat, w, b):
    nb = (2 * NU) // RB
    half = nb // 2
    return pl.pallas_call(
        _tc_transform_body,
        grid=(nb,),
        in_specs=[
            pl.BlockSpec((RB, DIM), lambda j: (j, 0)),
            pl.BlockSpec((1, RB, DIM), lambda j: (j // 10, j % 10, 0)),
            pl.BlockSpec((1, RB, DIM), lambda j: (2 + j // 10, j % 10, 0)),
            pl.BlockSpec((1, 5, DIM, DIM), lambda j: (j // 10, 0, 0, 0)),
            pl.BlockSpec((1, 1, DIM), lambda j: (j // 10, 0, 0)),
        ],
        out_specs=pl.BlockSpec((RB, DIM), lambda j: (j, 0)),
        out_shape=jax.ShapeDtypeStruct((2 * NU, DIM), jnp.float32),
    )(ecat, zcat, zcat, w, b)


LB = 1024
NLB = NB // LB


def _tc_loss_body(u_ref, i_ref, y_ref, logit_ref, acc_ref):
    j = pl.program_id(0)
    u = u_ref[...]
    iv = i_ref[...]
    nrm = jnp.sqrt(jnp.sum(u * u, axis=1, keepdims=True))
    un = u / jnp.maximum(nrm, 1e-12)
    lg = jnp.sum(un * iv, axis=1)
    y = y_ref[0, 0, :]
    per = jnp.maximum(lg, 0.0) - lg * y + jnp.log1p(jnp.exp(-jnp.abs(lg)))
    bce = jnp.sum(per)
    reg = jnp.sum(un * un) + jnp.sum(iv * iv)
    lanes = lax.broadcasted_iota(jnp.int32, (1, 128), 1)
    row = jnp.where(lanes == 0, bce, jnp.where(lanes == 1, reg, 0.0))
    logit_ref[0, 0, :] = lg

    @pl.when(j == 0)
    def _():
        acc_ref[...] = row

    @pl.when(j > 0)
    def _():
        acc_ref[...] += row


def _tc_loss(rows, y3):
    return pl.pallas_call(
        _tc_loss_body,
        grid=(NLB,),
        in_specs=[
            pl.BlockSpec((LB, DIM), lambda j: (j, 0)),
            pl.BlockSpec((LB, DIM), lambda j: (j + NLB, 0)),
            pl.BlockSpec((1, 1, LB), lambda j: (j, 0, 0)),
        ],
        out_specs=[
            pl.BlockSpec((1, 1, LB), lambda j: (j, 0, 0)),
            pl.BlockSpec((1, 128), lambda j: (0, 0)),
        ],
        out_shape=[
            jax.ShapeDtypeStruct((NLB, 1, LB), jnp.float32),
            jax.ShapeDtypeStruct((1, 128), jnp.float32),
        ],
    )(rows, rows, y3)


def kernel(E_u_0, E_i_0, Wu, bu, Wi, bi, pos_values, neg_values,
           pos_edge_index, neg_edge_index, uids, iids, labels):
    pr = pos_edge_index[0].astype(jnp.int32)
    pc = pos_edge_index[1].astype(jnp.int32)
    nr = neg_edge_index[0].astype(jnp.int32)
    nc = neg_edge_index[1].astype(jnp.int32)
    srcs = jnp.concatenate([pc + NU, pr, nc + NU, nr])
    dsts = jnp.concatenate([pr, pc, nr, nc])
    vals = jnp.concatenate([pos_values, pos_values, neg_values, neg_values])
    wall = jnp.stack([Wu, Wi], axis=1)
    ball = jnp.stack([bu.sum(axis=1), bi.sum(axis=1)], axis=1)
    ball = ball.reshape(NLAYER, 2, 1, DIM)
    ecat = jnp.concatenate([E_u_0, E_i_0], axis=0)

    for l in range(NLAYER):
        zcat = _sc_spmm(ecat, srcs, dsts, vals)
        ecat = _tc_transform(ecat, zcat, wall[l], ball[l])

    gidx = jnp.concatenate([uids.astype(jnp.int32),
                            iids.astype(jnp.int32) + NU])
    rows = _sc_gather(ecat, gidx)
    y3 = labels.astype(jnp.float32).reshape(NLB, 1, LB)
    logit3, acc = _tc_loss(rows, y3)
    logits = logit3.reshape(NB)
    loss = acc[0, 0] / NB + 1e-6 * acc[0, 1]
    return (loss, logits)
